# baseline (speedup 1.0000x reference)
import functools

import jax
import jax.numpy as jnp
from jax import lax
from jax.experimental import pallas as pl
from jax.experimental.pallas import tpu as pltpu
from jax.experimental.pallas import tpu_sc as plsc

_E = 8
_K = 2
_BM = 512
_DK = 2048
_SCCH = 32


def _router_body(x_ref, gw_ref, e_ref, w_ref):
    logits = lax.dot_general(
        x_ref[...], gw_ref[...], (((1,), (1,)), ((), ())),
        preferred_element_type=jnp.float32)
    i1 = jnp.argmax(logits, axis=1, keepdims=True)
    l1 = jnp.max(logits, axis=1, keepdims=True)
    iota = lax.broadcasted_iota(jnp.int32, logits.shape, 1)
    masked = jnp.where(iota == i1, -jnp.inf, logits)
    i2 = jnp.argmax(masked, axis=1, keepdims=True)
    l2 = jnp.max(masked, axis=1, keepdims=True)
    w1 = 1.0 / (1.0 + jnp.exp(l2 - l1))
    e_ref[...] = jnp.concatenate([i1, i2], axis=1).astype(jnp.int32)
    w_ref[...] = jnp.concatenate([w1, 1.0 - w1], axis=1)


def _route(x2, gate_w):
    T = x2.shape[0]
    return pl.pallas_call(
        _router_body,
        out_shape=[jax.ShapeDtypeStruct((T, _K), jnp.int32),
                   jax.ShapeDtypeStruct((T, _K), jnp.float32)],
    )(x2, gate_w)


def _bookkeeping(e_pairs):
    T = e_pairs.shape[0]
    npair = T * _K
    e_flat = e_pairs.reshape(-1)
    onehot = (e_flat[:, None] == jnp.arange(_E, dtype=jnp.int32)[None, :])
    onehot = onehot.astype(jnp.int32)
    cum = jnp.cumsum(onehot, axis=0)
    rank = jnp.sum(cum * onehot, axis=1) - 1
    counts = cum[-1]
    nblk = (counts + _BM - 1) // _BM
    ends = jnp.cumsum(nblk)
    starts = ends - nblk
    row = (jnp.sum(starts[None, :] * onehot, axis=1) * _BM
           + rank).astype(jnp.int32)
    NB = npair // _BM + _E
    rows_pair = row.reshape(T, _K)
    bidx = jnp.arange(NB, dtype=jnp.int32)
    block_expert = jnp.minimum(
        jnp.sum((bidx[:, None] >= ends[None, :]).astype(jnp.int32), axis=1),
        _E - 1).astype(jnp.int32)
    used = ends[_E - 1:].astype(jnp.int32)
    return rows_pair, block_expert, used


@functools.lru_cache(maxsize=None)
def _make_dispatch(T, D, P):
    info = plsc.get_sparse_core_info()
    nw = info.num_cores * info.num_subcores
    tok_w = T // nw
    ch = _SCCH
    n_ch = tok_w // ch
    mesh = plsc.VectorSubcoreMesh(core_axis_name="c", subcore_axis_name="s")

    @functools.partial(
        pl.kernel,
        out_type=jax.ShapeDtypeStruct((P, D), jnp.float32),
        mesh=mesh,
        scratch_types=[
            pltpu.VMEM((n_ch, ch), jnp.int32),
            pltpu.VMEM((n_ch, ch), jnp.int32),
            pltpu.VMEM((2, ch, D), jnp.float32),
            pltpu.SemaphoreType.DMA,
            pltpu.SemaphoreType.DMA,
            pltpu.SemaphoreType.DMA,
        ],
        name="sc_dispatch_scatter",
    )
    def dispatch(x_hbm, r0_hbm, r1_hbm, xs_hbm, i0_v, i1_v, rows_v,
                 lsem, s0, s1):
        wid = lax.axis_index("s") * info.num_cores + lax.axis_index("c")
        base = wid * tok_w
        pltpu.sync_copy(r0_hbm.at[wid], i0_v)
        pltpu.sync_copy(r1_hbm.at[wid], i1_v)
        pltpu.async_copy(
            x_hbm.at[pl.ds(base, ch)], rows_v.at[0], lsem)
        waited = 0
        stores = []
        for c in range(n_ch):
            cur = c % 2
            pltpu.make_async_copy(
                x_hbm.at[pl.ds(base + c * ch, ch)], rows_v.at[cur],
                lsem).wait()
            stores.append(pltpu.async_copy(
                rows_v.at[cur], xs_hbm.at[i0_v.at[c]], s0))
            stores.append(pltpu.async_copy(
                rows_v.at[cur], xs_hbm.at[i1_v.at[c]], s1))
            if c + 1 < n_ch:
                if c >= 1:
                    stores[waited].wait()
                    stores[waited + 1].wait()
                    waited += 2
                pltpu.async_copy(
                    x_hbm.at[pl.ds(base + (c + 1) * ch, ch)],
                    rows_v.at[(c + 1) % 2], lsem)
        for cp in stores[waited:]:
            cp.wait()

    return dispatch


@functools.lru_cache(maxsize=None)
def _make_combine(T, D, P, kt):
    info = plsc.get_sparse_core_info()
    nw = info.num_cores * info.num_subcores
    tok_w = T // nw
    rch = 16
    n_ch = tok_w // rch
    mesh = plsc.VectorSubcoreMesh(core_axis_name="c", subcore_axis_name="s")

    @functools.partial(
        pl.kernel,
        out_type=[jax.ShapeDtypeStruct((kt * T, D), jnp.float32),
                  jax.ShapeDtypeStruct((kt * T, D), jnp.float32)],
        mesh=mesh,
        scratch_types=[
            pltpu.VMEM((kt * n_ch, rch), jnp.int32),
            pltpu.VMEM((kt * n_ch, rch), jnp.int32),
            pltpu.VMEM((2, rch, D), jnp.float32),
            pltpu.VMEM((2, rch, D), jnp.float32),
            pltpu.SemaphoreType.DMA,
            pltpu.SemaphoreType.DMA,
            pltpu.SemaphoreType.DMA,
            pltpu.SemaphoreType.DMA,
        ],
        name="sc_combine_gather",
    )
    def combine(ys_hbm, ia_hbm, ib_hbm, g0_hbm, g1_hbm,
                i0_v, i1_v, a_v, b_v, ga, gb, sa, sb):
        wid = lax.axis_index("s") * info.num_cores + lax.axis_index("c")
        tbase = wid * tok_w
        pltpu.sync_copy(ia_hbm.at[wid], i0_v)
        pltpu.sync_copy(ib_hbm.at[wid], i1_v)
        store = []
        for kk in range(kt):
            for c in range(n_ch):
                cur = c % 2
                ci = kk * n_ch + c
                obase = kk * T + tbase + c * rch
                cpa = pltpu.async_copy(
                    ys_hbm.at[i0_v.at[ci]], a_v.at[cur], ga)
                cpb = pltpu.async_copy(
                    ys_hbm.at[i1_v.at[ci]], b_v.at[cur], gb)
                cpa.wait()
                cpb.wait()
                store.append(pltpu.async_copy(
                    a_v.at[cur], g0_hbm.at[pl.ds(obase, rch)], sa))
                store.append(pltpu.async_copy(
                    b_v.at[cur], g1_hbm.at[pl.ds(obase, rch)], sb))
        for cp in store:
            cp.wait()

    return combine


def _add_body(a_ref, b_ref, w_ref, o_ref):
    kt = a_ref.shape[0]
    w0 = w_ref[:, 0:1]
    w1 = w_ref[:, 1:2]
    a = a_ref[0].astype(jnp.float32)
    b = b_ref[0].astype(jnp.float32)
    for j in range(1, kt):
        a = a + a_ref[j].astype(jnp.float32)
        b = b + b_ref[j].astype(jnp.float32)
    o_ref[...] = a * w0 + b * w1


def _tc_add(a, b, w_pairs):
    kt, T, D = a.shape
    bt = 256
    return pl.pallas_call(
        _add_body,
        grid=(T // bt,),
        in_specs=[pl.BlockSpec((kt, bt, D), lambda i: (0, i, 0)),
                  pl.BlockSpec((kt, bt, D), lambda i: (0, i, 0)),
                  pl.BlockSpec((bt, _K), lambda i: (i, 0))],
        out_specs=pl.BlockSpec((bt, D), lambda i: (i, 0)),
        out_shape=jax.ShapeDtypeStruct((T, D), jnp.float32),
    )(a, b, w_pairs)


def _gmm_body(be_ref, used_ref, xs_ref, fca_ref, fcb_ref, pja_ref, pjb_ref,
              out_ref):
    b = pl.program_id(1)
    valid = b < used_ref[0]

    @pl.when(valid)
    def _compute():
        xb = xs_ref[...].astype(jnp.bfloat16)
        y = None
        for fc_ref, pj_ref in ((fca_ref, pja_ref), (fcb_ref, pjb_ref)):
            h = lax.dot_general(
                xb, fc_ref[0].astype(jnp.bfloat16),
                (((1,), (1,)), ((), ())),
                preferred_element_type=jnp.float32)
            h = jnp.square(jnp.maximum(h, 0.0)).astype(jnp.bfloat16)
            c = lax.dot_general(
                h, pj_ref[0].astype(jnp.bfloat16), (((1,), (1,)), ((), ())),
                preferred_element_type=jnp.float32)
            y = c if y is None else y + c
        out_ref[0] = y

    @pl.when(jnp.logical_not(valid))
    def _zero():
        out_ref[...] = jnp.zeros_like(out_ref)


def _gmm(xs, fc_w, proj_w, block_expert, used):
    P, D = xs.shape
    dff = fc_w.shape[1]
    nb = P // _BM
    kt = dff // _DK
    dk2 = _DK // 2
    grid_spec = pltpu.PrefetchScalarGridSpec(
        num_scalar_prefetch=2,
        grid=(kt, nb),
        in_specs=[
            pl.BlockSpec((_BM, D), lambda k, b, be, u: (b, 0)),
            pl.BlockSpec((1, dk2, D), lambda k, b, be, u: (be[b], 2 * k, 0)),
            pl.BlockSpec((1, dk2, D),
                         lambda k, b, be, u: (be[b], 2 * k + 1, 0)),
            pl.BlockSpec((1, D, dk2), lambda k, b, be, u: (be[b], 0, 2 * k)),
            pl.BlockSpec((1, D, dk2),
                         lambda k, b, be, u: (be[b], 0, 2 * k + 1)),
        ],
        out_specs=pl.BlockSpec((1, _BM, D), lambda k, b, be, u: (k, b, 0)),
    )
    return pl.pallas_call(
        _gmm_body,
        grid_spec=grid_spec,
        out_shape=jax.ShapeDtypeStruct((kt, P, D), jnp.float32),
    )(block_expert, used, xs, fc_w, fc_w, proj_w, proj_w)


def kernel(x, gate_w, fc_w, proj_w):
    b, l, d = x.shape
    T = b * l
    x2 = x.reshape(T, d)
    e_pairs, w_pairs = _route(x2, gate_w)
    rows_pair, block_expert, used = _bookkeeping(e_pairs)
    NB = (T * _K) // _BM + _E
    P = NB * _BM
    info = plsc.get_sparse_core_info()
    nw = info.num_cores * info.num_subcores
    n_ch = (T // nw) // _SCCH
    r0 = rows_pair[:, 0]
    r1 = rows_pair[:, 1]
    xs = _make_dispatch(T, d, P)(
        x2, r0.reshape(nw, n_ch, _SCCH), r1.reshape(nw, n_ch, _SCCH))
    ys = _gmm(xs, fc_w, proj_w, block_expert, used)
    kt = ys.shape[0]
    koff = P * jnp.arange(kt, dtype=jnp.int32)
    tok_w = T // nw
    ia = (r0.reshape(nw, tok_w)[:, None, :] + koff[None, :, None]
          ).reshape(nw, -1, 16)
    ib = (r1.reshape(nw, tok_w)[:, None, :] + koff[None, :, None]
          ).reshape(nw, -1, 16)
    g0, g1 = _make_combine(T, d, P, kt)(ys.reshape(kt * P, d), ia, ib)
    out = _tc_add(g0.reshape(kt, T, d), g1.reshape(kt, T, d), w_pairs)
    return out.reshape(b, l, d)


# speedup vs baseline: 1.0192x; 1.0192x over previous
import functools

import jax
import jax.numpy as jnp
from jax import lax
from jax.experimental import pallas as pl
from jax.experimental.pallas import tpu as pltpu
from jax.experimental.pallas import tpu_sc as plsc

_E = 8
_K = 2
_BM = 512
_DK = 2048
_SCCH = 32


def _router_body(x_ref, gw_ref, rows_ref, w_ref, be_ref, used_ref):
    T = x_ref.shape[0]
    nb = be_ref.shape[0]
    logits = lax.dot_general(
        x_ref[...], gw_ref[...], (((1,), (1,)), ((), ())),
        preferred_element_type=jnp.float32)
    i1 = jnp.argmax(logits, axis=1, keepdims=True)
    l1 = jnp.max(logits, axis=1, keepdims=True)
    iota = lax.broadcasted_iota(jnp.int32, logits.shape, 1)
    masked = jnp.where(iota == i1, -jnp.inf, logits)
    i2 = jnp.argmax(masked, axis=1, keepdims=True)
    l2 = jnp.max(masked, axis=1, keepdims=True)
    w1 = 1.0 / (1.0 + jnp.exp(l2 - l1))
    w_ref[...] = jnp.concatenate([w1, 1.0 - w1], axis=1)

    oh0 = (iota == i1).astype(jnp.float32)
    oh1 = (iota == i2).astype(jnp.float32)
    s = oh0 + oh1
    ri = lax.broadcasted_iota(jnp.int32, (T, T), 0)
    ci = lax.broadcasted_iota(jnp.int32, (T, T), 1)
    tril = (ci <= ri).astype(jnp.bfloat16)
    cum = lax.dot_general(
        tril, s.astype(jnp.bfloat16), (((1,), (0,)), ((), ())),
        preferred_element_type=jnp.float32)
    cumx = cum - s
    counts = jnp.sum(s, axis=0, keepdims=True)
    nblk = jnp.floor((counts + (_BM - 1)) * (1.0 / _BM))
    ei = lax.broadcasted_iota(jnp.int32, (_E, _E), 0)
    ej = lax.broadcasted_iota(jnp.int32, (_E, _E), 1)
    triu = (ei <= ej).astype(jnp.float32)
    ends = lax.dot_general(
        nblk, triu, (((1,), (0,)), ((), ())),
        preferred_element_type=jnp.float32)
    starts = ends - nblk
    st0 = jnp.sum(starts * oh0, axis=1, keepdims=True)
    st1 = jnp.sum(starts * oh1, axis=1, keepdims=True)
    rank0 = jnp.sum(cumx * oh0, axis=1, keepdims=True)
    rank1 = jnp.sum(cumx * oh1, axis=1, keepdims=True)
    row0 = st0 * float(_BM) + rank0
    row1 = st1 * float(_BM) + rank1
    rows_ref[...] = jnp.concatenate([row0, row1], axis=1).astype(jnp.int32)
    bi = lax.broadcasted_iota(jnp.int32, (nb, _E), 0).astype(jnp.float32)
    be = jnp.sum((bi >= jnp.broadcast_to(ends, (nb, _E))).astype(jnp.float32),
                 axis=1, keepdims=True)
    be_ref[...] = jnp.minimum(be, float(_E - 1)).astype(jnp.int32)
    used_ref[...] = ends[:, _E - 1:].astype(jnp.int32)


def _route(x2, gate_w):
    T = x2.shape[0]
    nb = (T * _K) // _BM + _E
    return pl.pallas_call(
        _router_body,
        out_shape=[jax.ShapeDtypeStruct((T, _K), jnp.int32),
                   jax.ShapeDtypeStruct((T, _K), jnp.float32),
                   jax.ShapeDtypeStruct((nb, 1), jnp.int32),
                   jax.ShapeDtypeStruct((1, 1), jnp.int32)],
    )(x2, gate_w)


def _bookkeeping(e_pairs):
    T = e_pairs.shape[0]
    npair = T * _K
    e_flat = e_pairs.reshape(-1)
    onehot = (e_flat[:, None] == jnp.arange(_E, dtype=jnp.int32)[None, :])
    onehot = onehot.astype(jnp.int32)
    cum = jnp.cumsum(onehot, axis=0)
    rank = jnp.sum(cum * onehot, axis=1) - 1
    counts = cum[-1]
    nblk = (counts + _BM - 1) // _BM
    ends = jnp.cumsum(nblk)
    starts = ends - nblk
    row = (jnp.sum(starts[None, :] * onehot, axis=1) * _BM
           + rank).astype(jnp.int32)
    NB = npair // _BM + _E
    rows_pair = row.reshape(T, _K)
    bidx = jnp.arange(NB, dtype=jnp.int32)
    block_expert = jnp.minimum(
        jnp.sum((bidx[:, None] >= ends[None, :]).astype(jnp.int32), axis=1),
        _E - 1).astype(jnp.int32)
    used = ends[_E - 1:].astype(jnp.int32)
    return rows_pair, block_expert, used


@functools.lru_cache(maxsize=None)
def _make_dispatch(T, D, P):
    info = plsc.get_sparse_core_info()
    nw = info.num_cores * info.num_subcores
    tok_w = T // nw
    ch = _SCCH
    n_ch = tok_w // ch
    mesh = plsc.VectorSubcoreMesh(core_axis_name="c", subcore_axis_name="s")

    @functools.partial(
        pl.kernel,
        out_type=jax.ShapeDtypeStruct((P, D), jnp.float32),
        mesh=mesh,
        scratch_types=[
            pltpu.VMEM((n_ch, ch), jnp.int32),
            pltpu.VMEM((n_ch, ch), jnp.int32),
            pltpu.VMEM((2, ch, D), jnp.float32),
            pltpu.SemaphoreType.DMA,
            pltpu.SemaphoreType.DMA,
            pltpu.SemaphoreType.DMA,
        ],
        name="sc_dispatch_scatter",
    )
    def dispatch(x_hbm, r0_hbm, r1_hbm, xs_hbm, i0_v, i1_v, rows_v,
                 lsem, s0, s1):
        wid = lax.axis_index("s") * info.num_cores + lax.axis_index("c")
        base = wid * tok_w
        pltpu.sync_copy(r0_hbm.at[wid], i0_v)
        pltpu.sync_copy(r1_hbm.at[wid], i1_v)
        pltpu.async_copy(
            x_hbm.at[pl.ds(base, ch)], rows_v.at[0], lsem)
        waited = 0
        stores = []
        for c in range(n_ch):
            cur = c % 2
            pltpu.make_async_copy(
                x_hbm.at[pl.ds(base + c * ch, ch)], rows_v.at[cur],
                lsem).wait()
            stores.append(pltpu.async_copy(
                rows_v.at[cur], xs_hbm.at[i0_v.at[c]], s0))
            stores.append(pltpu.async_copy(
                rows_v.at[cur], xs_hbm.at[i1_v.at[c]], s1))
            if c + 1 < n_ch:
                if c >= 1:
                    stores[waited].wait()
                    stores[waited + 1].wait()
                    waited += 2
                pltpu.async_copy(
                    x_hbm.at[pl.ds(base + (c + 1) * ch, ch)],
                    rows_v.at[(c + 1) % 2], lsem)
        for cp in stores[waited:]:
            cp.wait()

    return dispatch


@functools.lru_cache(maxsize=None)
def _make_combine(T, D, P, kt):
    info = plsc.get_sparse_core_info()
    nw = info.num_cores * info.num_subcores
    tok_w = T // nw
    rch = 16
    n_ch = tok_w // rch
    mesh = plsc.VectorSubcoreMesh(core_axis_name="c", subcore_axis_name="s")

    @functools.partial(
        pl.kernel,
        out_type=[jax.ShapeDtypeStruct((kt * T, D), jnp.float32),
                  jax.ShapeDtypeStruct((kt * T, D), jnp.float32)],
        mesh=mesh,
        scratch_types=[
            pltpu.VMEM((kt * n_ch, rch), jnp.int32),
            pltpu.VMEM((kt * n_ch, rch), jnp.int32),
            pltpu.VMEM((2, rch, D), jnp.float32),
            pltpu.VMEM((2, rch, D), jnp.float32),
            pltpu.SemaphoreType.DMA,
            pltpu.SemaphoreType.DMA,
            pltpu.SemaphoreType.DMA,
            pltpu.SemaphoreType.DMA,
        ],
        name="sc_combine_gather",
    )
    def combine(ys_hbm, ia_hbm, ib_hbm, g0_hbm, g1_hbm,
                i0_v, i1_v, a_v, b_v, ga, gb, sa, sb):
        wid = lax.axis_index("s") * info.num_cores + lax.axis_index("c")
        tbase = wid * tok_w
        pltpu.sync_copy(ia_hbm.at[wid], i0_v)
        pltpu.sync_copy(ib_hbm.at[wid], i1_v)
        store = []
        for kk in range(kt):
            for c in range(n_ch):
                cur = c % 2
                ci = kk * n_ch + c
                obase = kk * T + tbase + c * rch
                cpa = pltpu.async_copy(
                    ys_hbm.at[i0_v.at[ci]], a_v.at[cur], ga)
                cpb = pltpu.async_copy(
                    ys_hbm.at[i1_v.at[ci]], b_v.at[cur], gb)
                cpa.wait()
                cpb.wait()
                store.append(pltpu.async_copy(
                    a_v.at[cur], g0_hbm.at[pl.ds(obase, rch)], sa))
                store.append(pltpu.async_copy(
                    b_v.at[cur], g1_hbm.at[pl.ds(obase, rch)], sb))
        for cp in store:
            cp.wait()

    return combine


def _add_body(a_ref, b_ref, w_ref, o_ref):
    kt = a_ref.shape[0]
    w0 = w_ref[:, 0:1]
    w1 = w_ref[:, 1:2]
    a = a_ref[0].astype(jnp.float32)
    b = b_ref[0].astype(jnp.float32)
    for j in range(1, kt):
        a = a + a_ref[j].astype(jnp.float32)
        b = b + b_ref[j].astype(jnp.float32)
    o_ref[...] = a * w0 + b * w1


def _tc_add(a, b, w_pairs):
    kt, T, D = a.shape
    bt = 256
    return pl.pallas_call(
        _add_body,
        grid=(T // bt,),
        in_specs=[pl.BlockSpec((kt, bt, D), lambda i: (0, i, 0)),
                  pl.BlockSpec((kt, bt, D), lambda i: (0, i, 0)),
                  pl.BlockSpec((bt, _K), lambda i: (i, 0))],
        out_specs=pl.BlockSpec((bt, D), lambda i: (i, 0)),
        out_shape=jax.ShapeDtypeStruct((T, D), jnp.float32),
    )(a, b, w_pairs)


def _gmm_body(be_ref, used_ref, xs_ref, fca_ref, fcb_ref, pja_ref, pjb_ref,
              out_ref):
    b = pl.program_id(1)
    valid = b < used_ref[0]

    @pl.when(valid)
    def _compute():
        xb = xs_ref[...].astype(jnp.bfloat16)
        y = None
        for fc_ref, pj_ref in ((fca_ref, pja_ref), (fcb_ref, pjb_ref)):
            h = lax.dot_general(
                xb, fc_ref[0].astype(jnp.bfloat16),
                (((1,), (1,)), ((), ())),
                preferred_element_type=jnp.float32)
            h = jnp.square(jnp.maximum(h, 0.0)).astype(jnp.bfloat16)
            c = lax.dot_general(
                h, pj_ref[0].astype(jnp.bfloat16), (((1,), (1,)), ((), ())),
                preferred_element_type=jnp.float32)
            y = c if y is None else y + c
        out_ref[0] = y

    @pl.when(jnp.logical_not(valid))
    def _zero():
        out_ref[...] = jnp.zeros_like(out_ref)


def _gmm(xs, fc_w, proj_w, block_expert, used):
    P, D = xs.shape
    dff = fc_w.shape[1]
    nb = P // _BM
    kt = dff // _DK
    dk2 = _DK // 2
    grid_spec = pltpu.PrefetchScalarGridSpec(
        num_scalar_prefetch=2,
        grid=(kt, nb),
        in_specs=[
            pl.BlockSpec((_BM, D), lambda k, b, be, u: (b, 0)),
            pl.BlockSpec((1, dk2, D), lambda k, b, be, u: (be[b], 2 * k, 0)),
            pl.BlockSpec((1, dk2, D),
                         lambda k, b, be, u: (be[b], 2 * k + 1, 0)),
            pl.BlockSpec((1, D, dk2), lambda k, b, be, u: (be[b], 0, 2 * k)),
            pl.BlockSpec((1, D, dk2),
                         lambda k, b, be, u: (be[b], 0, 2 * k + 1)),
        ],
        out_specs=pl.BlockSpec((1, _BM, D), lambda k, b, be, u: (k, b, 0)),
    )
    return pl.pallas_call(
        _gmm_body,
        grid_spec=grid_spec,
        out_shape=jax.ShapeDtypeStruct((kt, P, D), jnp.float32),
    )(block_expert, used, xs, fc_w, fc_w, proj_w, proj_w)


def kernel(x, gate_w, fc_w, proj_w):
    b, l, d = x.shape
    T = b * l
    x2 = x.reshape(T, d)
    rows_pair, w_pairs, be2, used2 = _route(x2, gate_w)
    NB = (T * _K) // _BM + _E
    P = NB * _BM
    block_expert = be2.reshape(NB)
    used = used2.reshape(1)
    info = plsc.get_sparse_core_info()
    nw = info.num_cores * info.num_subcores
    n_ch = (T // nw) // _SCCH
    r0 = rows_pair[:, 0]
    r1 = rows_pair[:, 1]
    xs = _make_dispatch(T, d, P)(
        x2, r0.reshape(nw, n_ch, _SCCH), r1.reshape(nw, n_ch, _SCCH))
    ys = _gmm(xs, fc_w, proj_w, block_expert, used)
    kt = ys.shape[0]
    koff = P * jnp.arange(kt, dtype=jnp.int32)
    tok_w = T // nw
    ia = (r0.reshape(nw, tok_w)[:, None, :] + koff[None, :, None]
          ).reshape(nw, -1, 16)
    ib = (r1.reshape(nw, tok_w)[:, None, :] + koff[None, :, None]
          ).reshape(nw, -1, 16)
    g0, g1 = _make_combine(T, d, P, kt)(ys.reshape(kt * P, d), ia, ib)
    out = _tc_add(g0.reshape(kt, T, d), g1.reshape(kt, T, d), w_pairs)
    return out.reshape(b, l, d)
